# baseline (speedup 1.0000x reference)
import math

import jax
import jax.numpy as jnp
from jax.experimental import pallas as pl
from jax.experimental.pallas import tpu as pltpu

B, N, F, H, T = 8, 512, 128, 128, 16
MASKED = -1.0
_INV_SQRT_H = 1.0 / math.sqrt(H)


def _edge_kernel(ns_ref, skip_ref, x_ref, adj_ref, a_ref,
                 we2h, be2h, wmu_pr, wu_pr_lo,
                 wmu_bf, wu_bf_lo, we12,
                 out_ref):
    b = pl.program_id(0)
    ns = ns_ref[b]
    sk = skip_ref[b]
    f32 = jnp.float32
    bf16 = jnp.bfloat16

    @pl.when(sk == 0)
    def _compute():
        x = x_ref[0]
        adj = adj_ref[0]

        z2 = jnp.tanh(jnp.dot(x, we2h[...], preferred_element_type=f32)
                      + be2h[...])
        z_pr = z2[:, 0:H]
        z_bf = z2[:, H:2 * H]

        zmu_pr = jnp.dot(z_pr, wmu_pr[...], preferred_element_type=f32)
        zm_pr = zmu_pr[:, 0:H]
        zu_pr = zmu_pr[:, H:2 * H]

        def pr_tail(m):
            msg = jnp.dot(adj, m, preferred_element_type=f32)
            return jnp.maximum(
                zu_pr + jnp.dot(msg, wu_pr_lo[...], preferred_element_type=f32),
                0.0)

        h0 = pr_tail(jnp.maximum(zm_pr, 0.0))

        def pr_step(_, h):
            hm = jnp.dot(h, wmu_pr[:, 0:H], preferred_element_type=f32)
            return pr_tail(jnp.maximum(zm_pr + hm, 0.0))

        h = jax.lax.fori_loop(1, ns, pr_step, h0)

        zmu_bf = jnp.dot(z_bf, wmu_bf[...], preferred_element_type=f32)
        m2 = jnp.maximum(
            zmu_bf[:, 0:H] + jnp.dot(h, wmu_bf[:, 0:H],
                                     preferred_element_type=f32), 0.0)
        msg2 = jnp.dot(adj, m2, preferred_element_type=f32)
        hb = jnp.maximum(
            zmu_bf[:, H:2 * H] + jnp.dot(msg2, wu_bf_lo[...],
                                         preferred_element_type=f32), 0.0)
        e12 = jnp.dot(hb, we12[...], preferred_element_type=f32)
        e1 = e12[:, 0:H]
        e2 = e12[:, H:2 * H]
        cand = jax.lax.dot_general(
            e1, e2, (((1,), (1,)), ((), ())),
            preferred_element_type=f32) * _INV_SQRT_H
        out_ref[0] = a_ref[0] * cand

    @pl.when(sk != 0)
    def _masked():
        out_ref[0] = jnp.full((N, N), MASKED, f32)


def kernel(x, adj, A, W_enc_pr, b_enc_pr, W_msg_pr, W_upd_pr, w_node_pr,
           We1_pr, We2_pr, W_enc_bfs, b_enc_bfs, W_msg_bfs, W_upd_bfs,
           w_node_bfs, We1_bfs, We2_bfs, phase_logits, lengths):
    del w_node_pr, We1_pr, We2_pr, w_node_bfs

    p = jnp.argmax(phase_logits, axis=-1).astype(jnp.int32)
    iv = jnp.arange(T, dtype=jnp.int32)[:, None]
    ln = lengths.astype(jnp.int32)[None, :]
    valid = (iv >= 1) & (ln > iv + 1) & (p == 0)
    i_star = jnp.max(jnp.where(valid, iv, 0), axis=0)
    reset = (p == 1) & (iv < i_star[None, :])
    j0 = jnp.max(jnp.where(reset, iv + 1, 0), axis=0)
    nsteps = i_star - j0 + 1
    skip = ((i_star == 0) & (p[0] == 1)).astype(jnp.int32)

    we2h = jnp.concatenate([W_enc_pr, W_enc_bfs], axis=1)
    be2h = jnp.concatenate([b_enc_pr, b_enc_bfs]).reshape(1, 2 * H)
    wmu_pr = jnp.concatenate([W_msg_pr, W_upd_pr[:H]], axis=1)
    wu_pr_lo = W_upd_pr[H:2 * H]
    wmu_bf = jnp.concatenate([W_msg_bfs, W_upd_bfs[:H]], axis=1)
    wu_bf_lo = W_upd_bfs[H:2 * H]
    we12 = jnp.concatenate([We1_bfs, We2_bfs], axis=1)

    smem = pl.BlockSpec(memory_space=pltpu.SMEM)
    full = lambda *shape: pl.BlockSpec(shape, lambda b: (0,) * len(shape))
    batched = lambda *shape: pl.BlockSpec((1,) + shape, lambda b: (b, 0, 0))

    out = pl.pallas_call(
        _edge_kernel,
        grid=(B,),
        in_specs=[
            smem, smem,
            batched(N, F), batched(N, N), batched(N, N),
            full(F, 2 * H), full(1, 2 * H), full(H, 2 * H), full(H, H),
            full(H, 2 * H), full(H, H), full(H, 2 * H),
        ],
        out_specs=batched(N, N),
        out_shape=jax.ShapeDtypeStruct((B, N, N), jnp.float32),
    )(nsteps, skip, x, adj, A,
      we2h, be2h, wmu_pr, wu_pr_lo, wmu_bf, wu_bf_lo, we12)
    return out


# speedup vs baseline: 1.1067x; 1.1067x over previous
import math

import jax
import jax.numpy as jnp
from jax.experimental import pallas as pl
from jax.experimental.pallas import tpu as pltpu

B, N, F, H, T = 8, 512, 128, 128, 16
MASKED = -1.0
_INV_SQRT_H = 1.0 / math.sqrt(H)


def _edge_kernel(ns_ref, skip_ref, x_ref, adj_ref, a_ref,
                 we2h, be2h, wm_pr, wu_pr,
                 wm_bf, wu_bf, we12,
                 out_ref):
    b = pl.program_id(0)
    ns = ns_ref[b]
    sk = skip_ref[b]
    f32 = jnp.float32

    @pl.when(sk == 0)
    def _compute():
        x = x_ref[0]
        adj = adj_ref[0]

        z2 = jnp.tanh(jnp.dot(x, we2h[...], preferred_element_type=f32)
                      + be2h[...])
        z_pr = z2[:, 0:H]
        z_bf = z2[:, H:2 * H]

        def net_step(z, h, wm, wu):
            m = jnp.maximum(jnp.dot(z + h, wm[...],
                                    preferred_element_type=f32), 0.0)
            msg = jnp.dot(adj, m, preferred_element_type=f32)
            zc = jnp.concatenate([z, msg], axis=1)
            return jnp.maximum(jnp.dot(zc, wu[...],
                                       preferred_element_type=f32), 0.0)

        h = jax.lax.fori_loop(
            0, ns, lambda i, hh: net_step(z_pr, hh, wm_pr, wu_pr),
            jnp.zeros((N, H), f32))

        hb = net_step(z_bf, h, wm_bf, wu_bf)

        e12 = jnp.dot(hb, we12[...], preferred_element_type=f32)
        e1 = e12[:, 0:H]
        e2 = e12[:, H:2 * H]
        cand = jax.lax.dot_general(
            e1, e2, (((1,), (1,)), ((), ())),
            preferred_element_type=f32) * _INV_SQRT_H
        out_ref[0] = a_ref[0] * cand

    @pl.when(sk != 0)
    def _masked():
        out_ref[0] = jnp.full((N, N), MASKED, f32)


def kernel(x, adj, A, W_enc_pr, b_enc_pr, W_msg_pr, W_upd_pr, w_node_pr,
           We1_pr, We2_pr, W_enc_bfs, b_enc_bfs, W_msg_bfs, W_upd_bfs,
           w_node_bfs, We1_bfs, We2_bfs, phase_logits, lengths):
    del w_node_pr, We1_pr, We2_pr, w_node_bfs

    p = jnp.argmax(phase_logits, axis=-1).astype(jnp.int32)
    iv = jnp.arange(T, dtype=jnp.int32)[:, None]
    ln = lengths.astype(jnp.int32)[None, :]
    valid = (iv >= 1) & (ln > iv + 1) & (p == 0)
    i_star = jnp.max(jnp.where(valid, iv, 0), axis=0)
    reset = (p == 1) & (iv < i_star[None, :])
    j0 = jnp.max(jnp.where(reset, iv + 1, 0), axis=0)
    nsteps = i_star - j0 + 1
    skip = ((i_star == 0) & (p[0] == 1)).astype(jnp.int32)

    we2h = jnp.concatenate([W_enc_pr, W_enc_bfs], axis=1)
    be2h = jnp.concatenate([b_enc_pr, b_enc_bfs]).reshape(1, 2 * H)
    we12 = jnp.concatenate([We1_bfs, We2_bfs], axis=1)

    smem = pl.BlockSpec(memory_space=pltpu.SMEM)
    full = lambda *shape: pl.BlockSpec(shape, lambda b: (0,) * len(shape))
    batched = lambda *shape: pl.BlockSpec((1,) + shape, lambda b: (b, 0, 0))

    out = pl.pallas_call(
        _edge_kernel,
        grid=(B,),
        in_specs=[
            smem, smem,
            batched(N, F), batched(N, N), batched(N, N),
            full(F, 2 * H), full(1, 2 * H), full(H, H), full(2 * H, H),
            full(H, H), full(2 * H, H), full(H, 2 * H),
        ],
        out_specs=batched(N, N),
        out_shape=jax.ShapeDtypeStruct((B, N, N), jnp.float32),
    )(nsteps, skip, x, adj, A,
      we2h, be2h, W_msg_pr, W_upd_pr, W_msg_bfs, W_upd_bfs, we12)
    return out
